# Initial kernel scaffold; baseline (speedup 1.0000x reference)
import jax
import jax.numpy as jnp
from jax.experimental import pallas as pl


def kernel(input, edge_index, edge_weight, W):
    raise NotImplementedError("write your pallas kernel here")


# speedup vs baseline: 2.9123x; 2.9123x over previous
import functools

import jax
import jax.numpy as jnp
from jax import lax
from jax.experimental import pallas as pl
from jax.experimental.pallas import tpu as pltpu
from jax.experimental.pallas import tpu_sc as plsc

N = 10000
E = 160000
D = 256
H = 128
NS = 16
LANES = 16
EC = 128
NCHUNK = 80
EPT = EC * NCHUNK
E_PAD = EPT * NS
GRP = EC // LANES
ACC_ROWS = N + 8


def _sc_aggregate(xL, xR, dst3, col3, w3):
    mesh = plsc.VectorSubcoreMesh(core_axis_name="c", subcore_axis_name="s")

    @functools.partial(
        pl.kernel,
        out_type=[jax.ShapeDtypeStruct((N, H), jnp.float32),
                  jax.ShapeDtypeStruct((N, H), jnp.float32)],
        mesh=mesh,
        scratch_types=[
            pltpu.VMEM((NCHUNK, EC), jnp.int32),
            pltpu.VMEM((NCHUNK, EC), jnp.int32),
            pltpu.VMEM((NCHUNK, EC), jnp.float32),
            pltpu.VMEM((EC, H), jnp.float32),
            pltpu.VMEM_SHARED((ACC_ROWS, H), jnp.float32),
            pltpu.SemaphoreType.DMA,
        ],
    )
    def k(xLh, xRh, dst_h, col_h, w_h, outL, outR,
          col_v, dst_v, w_v, buf, acc, sem):
        c = lax.axis_index("c")
        s = lax.axis_index("s")
        pltpu.sync_copy(col_h.at[s], col_v)
        pltpu.sync_copy(dst_h.at[s], dst_v)
        pltpu.sync_copy(w_h.at[s], w_v)

        zv = jnp.zeros((LANES,), jnp.float32)

        def zero_buf_rows(nrows):
            def zrow(e, _):
                for d2 in range(H // LANES):
                    buf[e, pl.ds(d2 * LANES, LANES)] = zv
                return 0
            lax.fori_loop(0, nrows, zrow, 0)

        zero_buf_rows(EC)
        for t in range(NS):
            @pl.when(s == t)
            def _(t=t):
                b = 624 * t
                ln = 648 if t == NS - 1 else 624
                for i in range(ln // EC):
                    pltpu.sync_copy(buf, acc.at[pl.ds(b + i * EC, EC)])
                tail = ln % EC
                if tail:
                    pltpu.sync_copy(buf.at[pl.ds(0, tail)],
                                    acc.at[pl.ds(b + (ln // EC) * EC, tail)])
        plsc.subcore_barrier()

        def run(xh):
            def chunk(j, _):
                pltpu.async_copy(xh.at[col_v.at[j]], buf, sem).wait()

                def group(g, _):
                    w16 = w_v[j, pl.ds(g * LANES, LANES)]
                    for l in range(LANES):
                        e = g * LANES + l
                        w = w16[l]
                        for d2 in range(H // LANES):
                            sl = pl.ds(d2 * LANES, LANES)
                            buf[e, sl] = buf[e, sl] * w
                    return 0
                lax.fori_loop(0, GRP, group, 0)
                pltpu.sync_copy(buf, acc.at[dst_v.at[j]], add=True)
                return 0
            lax.fori_loop(0, NCHUNK, chunk, 0)

        def flush(out_hbm):
            for t in range(NS):
                @pl.when(s == t)
                def _(t=t):
                    b = 624 * t
                    ln = 640 if t == NS - 1 else 624
                    pltpu.sync_copy(acc.at[pl.ds(b, ln)],
                                    out_hbm.at[pl.ds(b, ln)])

        @pl.when(c == 0)
        def _():
            run(xLh)

        @pl.when(c == 1)
        def _():
            run(xRh)

        plsc.subcore_barrier()

        @pl.when(c == 0)
        def _():
            flush(outL)

        @pl.when(c == 1)
        def _():
            flush(outR)

    return k(xL, xR, dst3, col3, w3)


def _tc_matmul_relu(aL, aR, Wt, Wb):
    BM = 1000

    def body(aL_ref, aR_ref, wt_ref, wb_ref, o_ref):
        acc = jnp.dot(aL_ref[...], wt_ref[...],
                      preferred_element_type=jnp.float32,
                      precision=lax.Precision.HIGHEST)
        acc = acc + jnp.dot(aR_ref[...], wb_ref[...],
                            preferred_element_type=jnp.float32,
                            precision=lax.Precision.HIGHEST)
        o_ref[...] = jnp.maximum(acc, 0.0)

    return pl.pallas_call(
        body,
        grid=(N // BM,),
        in_specs=[pl.BlockSpec((BM, H), lambda i: (i, 0)),
                  pl.BlockSpec((BM, H), lambda i: (i, 0)),
                  pl.BlockSpec((H, D), lambda i: (0, 0)),
                  pl.BlockSpec((H, D), lambda i: (0, 0))],
        out_specs=pl.BlockSpec((BM, D), lambda i: (i, 0)),
        out_shape=jax.ShapeDtypeStruct((N, D), jnp.float32),
    )(aL, aR, Wt, Wb)


def kernel(input, edge_index, edge_weight, W):
    ei = edge_index.astype(jnp.int32)
    npad = E_PAD - E
    dst = jnp.concatenate([ei[0], jnp.full((npad,), N, jnp.int32)])
    col = jnp.concatenate([ei[1], jnp.zeros((npad,), jnp.int32)])
    ew = jnp.concatenate([edge_weight, jnp.zeros((npad,), jnp.float32)])
    dst3 = dst.reshape(NS, NCHUNK, EC)
    col3 = col.reshape(NS, NCHUNK, EC)
    w3 = ew.reshape(NS, NCHUNK, EC)
    xL = input[:, :H]
    xR = input[:, H:]
    aggL, aggR = _sc_aggregate(xL, xR, dst3, col3, w3)
    return _tc_matmul_relu(aggL, aggR, W[:H], W[H:])


# speedup vs baseline: 3.6564x; 1.2555x over previous
import functools

import jax
import jax.numpy as jnp
from jax import lax
from jax.experimental import pallas as pl
from jax.experimental.pallas import tpu as pltpu
from jax.experimental.pallas import tpu_sc as plsc

N = 10000
E = 160000
D = 256
H = 128
NS = 16
LANES = 16
EC = 128
NCHUNK = 80
NPAIR = NCHUNK // 2
EPT = EC * NCHUNK
E_PAD = EPT * NS
GRP = EC // LANES


def _sc_aggregate(xL, xR, dst2, col3, w2):
    mesh = plsc.VectorSubcoreMesh(core_axis_name="c", subcore_axis_name="s")

    @functools.partial(
        pl.kernel,
        out_type=[jax.ShapeDtypeStruct((N, H), jnp.float32),
                  jax.ShapeDtypeStruct((N, H), jnp.float32)],
        mesh=mesh,
        scratch_types=[
            pltpu.VMEM((NCHUNK, EC), jnp.int32),
            pltpu.VMEM((2, EC), jnp.int32),
            pltpu.VMEM((2, EC), jnp.float32),
            pltpu.VMEM((2, EC, H), jnp.float32),
            pltpu.VMEM_SHARED((N, H), jnp.float32),
            pltpu.SemaphoreType.DMA,
            pltpu.SemaphoreType.DMA,
            pltpu.SemaphoreType.DMA,
            pltpu.SemaphoreType.DMA,
            pltpu.SemaphoreType.DMA,
            pltpu.SemaphoreType.DMA,
        ],
    )
    def k(xLh, xRh, dst_h, col_h, w_h, outL, outR,
          col_v, dring, wring, buf, acc,
          gsem0, gsem1, ssem0, ssem1, dwsem0, dwsem1):
        c = lax.axis_index("c")
        s = lax.axis_index("s")
        pltpu.sync_copy(col_h.at[s], col_v)
        base = s * NCHUNK

        zv = jnp.zeros((LANES,), jnp.float32)

        def zrow(e, _):
            for d2 in range(H // LANES):
                buf[0, e, pl.ds(d2 * LANES, LANES)] = zv
            return 0
        lax.fori_loop(0, EC, zrow, 0)
        for t in range(NS):
            @pl.when(s == t)
            def _(t=t):
                b = 624 * t
                ln = 640 if t == NS - 1 else 624
                for i in range(ln // EC):
                    pltpu.sync_copy(buf.at[0], acc.at[pl.ds(b + i * EC, EC)])
                tail = ln % EC
                if tail:
                    pltpu.sync_copy(
                        buf.at[0, pl.ds(0, tail)],
                        acc.at[pl.ds(b + (ln // EC) * EC, tail)])
        plsc.subcore_barrier()

        def run(xh):
            def scale(b):
                def group(g, _):
                    w16 = wring[b, pl.ds(g * LANES, LANES)]
                    for l in range(LANES):
                        e = g * LANES + l
                        w = w16[l]
                        for d2 in range(H // LANES):
                            sl = pl.ds(d2 * LANES, LANES)
                            buf[b, e, sl] = buf[b, e, sl] * w
                    return 0
                lax.fori_loop(0, GRP, group, 0)

            def gather_wait(b, gsem):
                pltpu.make_async_copy(xh.at[col_v.at[0]], buf.at[b],
                                      gsem).wait()

            def dw_wait(b, dwsem):
                pltpu.make_async_copy(dst_h.at[0], dring.at[b], dwsem).wait()
                pltpu.make_async_copy(w_h.at[0], wring.at[b], dwsem).wait()

            def scatter_wait(b, ssem):
                pltpu.make_async_copy(buf.at[b], acc.at[dring.at[b]],
                                      ssem).wait()

            pltpu.sync_copy(dst_h.at[base], dring.at[0])
            pltpu.sync_copy(w_h.at[base], wring.at[0])
            pltpu.async_copy(xh.at[col_v.at[0]], buf.at[0], gsem0)

            def pair(g, _):
                j0 = 2 * g
                j1 = j0 + 1

                @pl.when(g > 0)
                def _():
                    scatter_wait(1, ssem1)
                pltpu.async_copy(dst_h.at[base + j1], dring.at[1], dwsem1)
                pltpu.async_copy(w_h.at[base + j1], wring.at[1], dwsem1)
                pltpu.async_copy(xh.at[col_v.at[j1]], buf.at[1], gsem1)

                gather_wait(0, gsem0)

                @pl.when(g > 0)
                def _():
                    dw_wait(0, dwsem0)
                scale(0)
                pltpu.async_copy(buf.at[0], acc.at[dring.at[0]], ssem0,
                                 add=True)

                gather_wait(1, gsem1)
                dw_wait(1, dwsem1)
                scale(1)

                scatter_wait(0, ssem0)

                @pl.when(g < NPAIR - 1)
                def _():
                    pltpu.async_copy(dst_h.at[base + j0 + 2], dring.at[0],
                                     dwsem0)
                    pltpu.async_copy(w_h.at[base + j0 + 2], wring.at[0],
                                     dwsem0)
                    pltpu.async_copy(xh.at[col_v.at[j0 + 2]], buf.at[0],
                                     gsem0)
                pltpu.async_copy(buf.at[1], acc.at[dring.at[1]], ssem1,
                                 add=True)
                return 0
            lax.fori_loop(0, NPAIR, pair, 0)
            scatter_wait(1, ssem1)

        def flush(out_hbm):
            for t in range(NS):
                @pl.when(s == t)
                def _(t=t):
                    b = 624 * t
                    ln = 640 if t == NS - 1 else 624
                    pltpu.sync_copy(acc.at[pl.ds(b, ln)],
                                    out_hbm.at[pl.ds(b, ln)])

        @pl.when(c == 0)
        def _():
            run(xLh)

        @pl.when(c == 1)
        def _():
            run(xRh)

        plsc.subcore_barrier()

        @pl.when(c == 0)
        def _():
            flush(outL)

        @pl.when(c == 1)
        def _():
            flush(outR)

    return k(xL, xR, dst2, col3, w2)


def _tc_matmul_relu(aL, aR, Wt, Wb):
    BM = 1000

    def body(aL_ref, aR_ref, wt_ref, wb_ref, o_ref):
        acc = jnp.dot(aL_ref[...], wt_ref[...],
                      preferred_element_type=jnp.float32,
                      precision=lax.Precision.HIGHEST)
        acc = acc + jnp.dot(aR_ref[...], wb_ref[...],
                            preferred_element_type=jnp.float32,
                            precision=lax.Precision.HIGHEST)
        o_ref[...] = jnp.maximum(acc, 0.0)

    return pl.pallas_call(
        body,
        grid=(N // BM,),
        in_specs=[pl.BlockSpec((BM, H), lambda i: (i, 0)),
                  pl.BlockSpec((BM, H), lambda i: (i, 0)),
                  pl.BlockSpec((H, D), lambda i: (0, 0)),
                  pl.BlockSpec((H, D), lambda i: (0, 0))],
        out_specs=pl.BlockSpec((BM, D), lambda i: (i, 0)),
        out_shape=jax.ShapeDtypeStruct((N, D), jnp.float32),
    )(aL, aR, Wt, Wb)


def kernel(input, edge_index, edge_weight, W):
    ei = edge_index.astype(jnp.int32)
    npad = E_PAD - E
    dst = jnp.concatenate([ei[0], jnp.zeros((npad,), jnp.int32)])
    col = jnp.concatenate([ei[1], jnp.zeros((npad,), jnp.int32)])
    ew = jnp.concatenate([edge_weight, jnp.zeros((npad,), jnp.float32)])
    dst2 = dst.reshape(NS * NCHUNK, EC)
    col3 = col.reshape(NS, NCHUNK, EC)
    w2 = ew.reshape(NS * NCHUNK, EC)
    xL = input[:, :H]
    xR = input[:, H:]
    aggL, aggR = _sc_aggregate(xL, xR, dst2, col3, w2)
    return _tc_matmul_relu(aggL, aggR, W[:H], W[H:])
